# Initial kernel scaffold; baseline (speedup 1.0000x reference)
import jax
import jax.numpy as jnp
from jax.experimental import pallas as pl


def kernel(adj_idx_0, adj_val_0, adj_idx_1, adj_val_1, adj_idx_2, adj_val_2, user_embedding, item_embedding, criterion_embedding, w_gcn, W_gc_0, W_gc_1, W_rel_0, W_rel_1, trans_s1, trans_s2):
    raise NotImplementedError("write your pallas kernel here")


# speedup vs baseline: 1.8472x; 1.8472x over previous
import dataclasses

import jax
import jax.numpy as jnp
from jax import lax
from jax.experimental import pallas as pl
from jax.experimental.pallas import tpu as pltpu
from jax.experimental.pallas import tpu_sc as plsc

N_USERS = 25000
N_ITEMS = 25000
N = N_USERS + N_ITEMS
E = 800000
EMB = 64
HALF = 32
ATT = 32
C = 3

NC = 2
NS = 16
CHUNK = 128
ROWS_PER_SUB = 400
EROWS = NS * ROWS_PER_SUB
EPAD = EROWS * CHUNK
ACC_PER_SUB = 3200
NPAD = NS * ACC_PER_SUB

_F32 = jnp.float32


def _leaky(x):
    return jnp.where(x >= 0, x, 0.3 * x)


def _sc_spmm_body(*refs):
    xs = refs[0:6]
    srcs = refs[6:9]
    dsts = refs[9:12]
    vals = refs[12:15]
    ys = refs[15:21]
    accum, sidx, didx, valv, rowsb, zbuf = refs[21:]

    c = lax.axis_index("c")
    s = lax.axis_index("s")

    zero16 = jnp.zeros((16,), _F32)

    @pl.loop(0, CHUNK)
    def _(r):
        zbuf[r, pl.ds(0, 16)] = zero16
        zbuf[r, pl.ds(16, 16)] = zero16

    def zero_accum():
        @pl.loop(0, ACC_PER_SUB // CHUNK)
        def _(t):
            pltpu.sync_copy(
                zbuf, accum.at[pl.ds(s * ACC_PER_SUB + t * CHUNK, CHUNK)])

    zero_accum()
    plsc.subcore_barrier()

    for i in range(C):
        def edge_phase(table, i=i):
            @pl.loop(0, ROWS_PER_SUB)
            def _(g):
                row = s * ROWS_PER_SUB + g
                pltpu.sync_copy(srcs[i].at[row], sidx.at[0])
                pltpu.sync_copy(vals[i].at[row], valv)
                pltpu.sync_copy(dsts[i].at[row], didx.at[0])
                pltpu.sync_copy(table.at[sidx.at[0]], rowsb)

                @pl.loop(0, CHUNK)
                def _(e):
                    v16 = plsc.load_gather(
                        valv, [jnp.full((16,), e, jnp.int32)])
                    rowsb[e, pl.ds(0, 16)] = rowsb[e, pl.ds(0, 16)] * v16
                    rowsb[e, pl.ds(16, 16)] = rowsb[e, pl.ds(16, 16)] * v16

                pltpu.sync_copy(rowsb, accum.at[didx.at[0]], add=True)

        @pl.when(c == 0)
        def _(i=i):
            edge_phase(xs[2 * i])

        @pl.when(c == 1)
        def _(i=i):
            edge_phase(xs[2 * i + 1])

        plsc.subcore_barrier()

        sl = pl.ds(s * ACC_PER_SUB, ACC_PER_SUB)

        @pl.when(c == 0)
        def _(i=i, sl=sl):
            pltpu.sync_copy(accum.at[sl], ys[2 * i].at[sl])

        @pl.when(c == 1)
        def _(i=i, sl=sl):
            pltpu.sync_copy(accum.at[sl], ys[2 * i + 1].at[sl])

        if i < C - 1:
            zero_accum()
        plsc.subcore_barrier()


def _sc_compiler_params():
    cp = pltpu.CompilerParams()
    fields = pltpu.CompilerParams.__dataclass_fields__
    if "needs_layout_passes" in fields:
        cp = dataclasses.replace(cp, needs_layout_passes=False)
    if "use_tc_tiling_on_sc" in fields:
        cp = dataclasses.replace(cp, use_tc_tiling_on_sc=False)
    return cp


def _sc_spmm3(tables, srcs, dsts, vals):
    mesh = plsc.VectorSubcoreMesh(
        core_axis_name="c", subcore_axis_name="s",
        num_cores=NC, num_subcores=NS)
    out_ty = tuple(
        jax.ShapeDtypeStruct((NPAD, HALF), _F32) for _ in range(6))
    fn = pl.kernel(
        _sc_spmm_body,
        out_type=out_ty,
        mesh=mesh,
        scratch_types=[
            pltpu.VMEM_SHARED((NPAD, HALF), _F32),
            pltpu.VMEM((1, CHUNK), jnp.int32),
            pltpu.VMEM((1, CHUNK), jnp.int32),
            pltpu.VMEM((CHUNK,), _F32),
            pltpu.VMEM((CHUNK, HALF), _F32),
            pltpu.VMEM((CHUNK, HALF), _F32),
        ],
        name="sc_spmm3",
        compiler_params=_sc_compiler_params(),
    )
    return fn(*tables, *srcs, *dsts, *vals)


_BLK = 2000
_GRID = N // _BLK


def _make_tc_layer(final):
    def body(*refs):
        if final:
            (e0, e1, e2, p00, p01, p02, p10, p11, p12,
             wg, wgc, ce, s1c, s2c) = refs[:14]
            outs = refs[14:]
            p0 = (p00, p01, p02)
            p1h = (p10, p11, p12)
        else:
            e0, e1, e2, wg, wgc, ce, s1c, s2c = refs[:8]
            outs = refs[8:]

        es = (e0, e1, e2)
        wgv = wg[...]
        wgcv = wgc[...]
        cev = ce[...]
        s1v = s1c[...]
        s2v = s2c[...]

        g = []
        for cc in range(C):
            h = jnp.dot(es[cc][...], wgv, preferred_element_type=_F32)
            h = h * cev[cc][None, :]
            h = _leaky(jnp.dot(h, wgcv, preferred_element_type=_F32))
            g.append(h)

        t = [jnp.tanh(jnp.dot(g[cc], s1v, preferred_element_type=_F32))
             for cc in range(C)]

        for i in range(C):
            s2row = s2v[i][None, :]
            l = [jnp.sum(t[cc][:, 32 * i:32 * i + 32] * s2row,
                         axis=1, keepdims=True) for cc in range(C)]
            m = jnp.maximum(jnp.maximum(l[0], l[1]), l[2])
            w = [jnp.exp(x - m) for x in l]
            tot = w[0] + w[1] + w[2]
            out = _leaky((w[0] * g[0] + w[1] * g[1] + w[2] * g[2]) / tot)
            if final:
                acc = (p0[i][...] + p1h[i][...] + out) * (1.0 / 3.0)
                outs[i][...] = acc
            else:
                outs[2 * i][...] = out[:, :HALF]
                outs[2 * i + 1][...] = out[:, HALF:]

    row_spec = lambda w: pl.BlockSpec((_BLK, w), lambda b: (b, 0))
    wt_spec = lambda a, b_: pl.BlockSpec((a, b_), lambda b: (0, 0))

    if final:
        in_specs = ([row_spec(EMB)] * 3 + [row_spec(EMB)] * 3
                    + [row_spec(EMB)] * 3
                    + [wt_spec(EMB, EMB), wt_spec(EMB, EMB),
                       wt_spec(8, EMB), wt_spec(EMB, 96), wt_spec(8, 32)])
        out_specs = [row_spec(EMB)] * 3
        out_shape = [jax.ShapeDtypeStruct((N, EMB), _F32)] * 3
    else:
        in_specs = ([row_spec(EMB)] * 3
                    + [wt_spec(EMB, EMB), wt_spec(EMB, EMB),
                       wt_spec(8, EMB), wt_spec(EMB, 96), wt_spec(8, 32)])
        out_specs = [row_spec(HALF)] * 6
        out_shape = [jax.ShapeDtypeStruct((N, HALF), _F32)] * 6

    return pl.pallas_call(
        body,
        grid=(_GRID,),
        in_specs=in_specs,
        out_specs=out_specs,
        out_shape=out_shape,
    )


def _cri_chain(ce0pad, wr0, wr1):
    def body(ce_ref, w0_ref, w1_ref, c1_ref, c2_ref, cm_ref):
        c0 = ce_ref[...]
        c1 = _leaky(jnp.dot(c0, w0_ref[...], preferred_element_type=_F32))
        c2 = _leaky(jnp.dot(c1, w1_ref[...], preferred_element_type=_F32))
        c1_ref[...] = c1
        c2_ref[...] = c2
        cm_ref[...] = (c0 + c1 + c2) * (1.0 / 3.0)

    return pl.pallas_call(
        body,
        out_shape=[jax.ShapeDtypeStruct((8, EMB), _F32)] * 3,
    )(ce0pad, wr0, wr1)


def _prep_edges(adj_idx, adj_val):
    pad = EPAD - E
    src = jnp.pad(adj_idx[1], (0, pad)).reshape(EROWS, CHUNK)
    dst = jnp.pad(adj_idx[0], (0, pad)).reshape(EROWS, CHUNK)
    val = jnp.pad(adj_val, (0, pad)).reshape(EROWS, CHUNK)
    return src, dst, val


def kernel(adj_idx_0, adj_val_0, adj_idx_1, adj_val_1, adj_idx_2, adj_val_2,
           user_embedding, item_embedding, criterion_embedding, w_gcn,
           W_gc_0, W_gc_1, W_rel_0, W_rel_1, trans_s1, trans_s2):
    srcs, dsts, vals = zip(*(
        _prep_edges(ai, av) for ai, av in
        ((adj_idx_0, adj_val_0), (adj_idx_1, adj_val_1),
         (adj_idx_2, adj_val_2))))

    p0 = [jnp.concatenate(
        [user_embedding[:, i, :], item_embedding[:, i, :]], axis=0)
        for i in range(C)]
    tables0 = []
    for i in range(C):
        tables0 += [p0[i][:, :HALF], p0[i][:, HALF:]]

    ce0 = jnp.pad(criterion_embedding, ((0, 8 - C), (0, 0)))
    c1p, _c2p, cmp_ = _cri_chain(ce0, W_rel_0, W_rel_1)
    s1c = jnp.concatenate([trans_s1[i] for i in range(C)], axis=1)
    s2c = jnp.pad(jnp.squeeze(trans_s2, -1), ((0, 8 - C), (0, 0)))

    layer1 = _make_tc_layer(final=False)
    layer2 = _make_tc_layer(final=True)

    y1 = _sc_spmm3(tables0, srcs, dsts, vals)
    e1 = [jnp.concatenate([y1[2 * i][:N], y1[2 * i + 1][:N]], axis=1)
          for i in range(C)]
    p1h = layer1(e1[0], e1[1], e1[2], w_gcn, W_gc_0, ce0, s1c, s2c)

    y2 = _sc_spmm3(list(p1h), srcs, dsts, vals)
    e2 = [jnp.concatenate([y2[2 * i][:N], y2[2 * i + 1][:N]], axis=1)
          for i in range(C)]
    p1full = [jnp.concatenate([p1h[2 * i], p1h[2 * i + 1]], axis=1)
              for i in range(C)]
    accs = layer2(e2[0], e2[1], e2[2],
                  p0[0], p0[1], p0[2],
                  p1full[0], p1full[1], p1full[2],
                  w_gcn, W_gc_1, c1p, s1c, s2c)

    acc = jnp.stack(accs, axis=1)
    users = acc[:N_USERS]
    items = jnp.concatenate(
        [acc[N_USERS:], jnp.zeros((1, C, EMB), _F32)], axis=0)
    cris = tuple(cmp_[i:i + 1] for i in range(C))
    return (users, items) + cris


# speedup vs baseline: 3.1865x; 1.7250x over previous
import dataclasses

import jax
import jax.numpy as jnp
from jax import lax
from jax.experimental import pallas as pl
from jax.experimental.pallas import tpu as pltpu
from jax.experimental.pallas import tpu_sc as plsc

N_USERS = 25000
N_ITEMS = 25000
N = N_USERS + N_ITEMS
E = 800000
EMB = 64
HALF = 32
ATT = 32
C = 3

NC = 2
NS = 16
CHUNK = 128
ROWS_PER_SUB = 400
EROWS = NS * ROWS_PER_SUB
EPAD = EROWS * CHUNK
ACC_PER_SUB = 3200
NPAD = NS * ACC_PER_SUB
G = 2
NG = ROWS_PER_SUB // G
ZROWS = 64

_F32 = jnp.float32


def _leaky(x):
    return jnp.where(x >= 0, x, 0.3 * x)


def _sc_spmm_body(xall, src_hbm, dst_hbm, val_hbm, yall,
                  accum, sidx, didx, valv, rowsb, zbuf, gsem, ssem, isem):
    c = lax.axis_index("c")
    s = lax.axis_index("s")
    coff16 = jnp.full((16,), c * (C * N), jnp.int32)

    zero16 = jnp.zeros((16,), _F32)

    @pl.loop(0, ZROWS)
    def _(r):
        zbuf[r, pl.ds(0, 16)] = zero16
        zbuf[r, pl.ds(16, 16)] = zero16

    def zero_accum():
        @pl.loop(0, ACC_PER_SUB // ZROWS)
        def _(t):
            pltpu.sync_copy(
                zbuf, accum.at[pl.ds(s * ACC_PER_SUB + t * ZROWS, ZROWS)])

    zero_accum()
    plsc.subcore_barrier()

    @pl.loop(0, C)
    def _(i):
        base = i * EROWS + s * ROWS_PER_SUB

        def load_idx(g, slot):
            dsl = pl.ds(slot * G, G)
            ssl = pl.ds(base + g * G, G)
            d1 = pltpu.async_copy(src_hbm.at[ssl], sidx.at[dsl], isem)
            d2 = pltpu.async_copy(dst_hbm.at[ssl], didx.at[dsl], isem)
            d3 = pltpu.async_copy(val_hbm.at[ssl], valv.at[dsl], isem)
            d1.wait()
            d2.wait()
            d3.wait()
            for j in range(G):
                r = slot * G + j

                @pl.loop(0, CHUNK, step=16)
                def _(k):
                    sidx[r, pl.ds(k, 16)] = sidx[r, pl.ds(k, 16)] + coff16

        def fire_gathers(slot):
            for j in range(G):
                r = slot * G + j
                pltpu.async_copy(
                    xall.at[sidx.at[r]], rowsb.at[r], gsem.at[slot])

        def drain(sem_slot):
            for j in range(G):
                pltpu.make_async_copy(
                    xall.at[pl.ds(0, CHUNK)], rowsb.at[j], sem_slot).wait()

        def scale(slot):
            for j in range(G):
                r = slot * G + j

                @pl.loop(0, CHUNK, unroll=4)
                def _(e):
                    v16 = plsc.load_gather(
                        valv,
                        [jnp.full((16,), r, jnp.int32),
                         jnp.full((16,), e, jnp.int32)])
                    rowsb[r, e, pl.ds(0, 16)] = (
                        rowsb[r, e, pl.ds(0, 16)] * v16)
                    rowsb[r, e, pl.ds(16, 16)] = (
                        rowsb[r, e, pl.ds(16, 16)] * v16)

        def fire_scatters(slot):
            for j in range(G):
                r = slot * G + j
                pltpu.async_copy(
                    rowsb.at[r], accum.at[didx.at[r]],
                    ssem.at[slot], add=True)

        def group_step(g, slot, first, maybe_last):
            other = 1 - slot
            drain(gsem.at[slot])
            scale(slot)
            if not first:
                drain(ssem.at[other])
            fire_scatters(slot)
            if maybe_last:
                @pl.when(g + 1 < NG)
                def _():
                    load_idx(g + 1, other)
                    fire_gathers(other)
            else:
                load_idx(g + 1, other)
                fire_gathers(other)

        load_idx(0, 0)
        fire_gathers(0)
        group_step(0, 0, True, False)

        @pl.loop(1, NG - 1, step=2)
        def _(g):
            group_step(g, 1, False, False)
            group_step(g + 1, 0, False, False)

        group_step(NG - 1, 1, False, True)
        drain(ssem.at[1])

        plsc.subcore_barrier()
        off = (c * C + i) * NPAD + s * ACC_PER_SUB
        pltpu.sync_copy(
            accum.at[pl.ds(s * ACC_PER_SUB, ACC_PER_SUB)],
            yall.at[pl.ds(off, ACC_PER_SUB)])
        zero_accum()
        plsc.subcore_barrier()


def _sc_compiler_params():
    cp = pltpu.CompilerParams()
    fields = pltpu.CompilerParams.__dataclass_fields__
    if "needs_layout_passes" in fields:
        cp = dataclasses.replace(cp, needs_layout_passes=False)
    if "use_tc_tiling_on_sc" in fields:
        cp = dataclasses.replace(cp, use_tc_tiling_on_sc=False)
    return cp


def _sc_spmm3(xall, src_cat, dst_cat, val_cat):
    mesh = plsc.VectorSubcoreMesh(
        core_axis_name="c", subcore_axis_name="s",
        num_cores=NC, num_subcores=NS)
    fn = pl.kernel(
        _sc_spmm_body,
        out_type=jax.ShapeDtypeStruct((NC * C * NPAD, HALF), _F32),
        mesh=mesh,
        scratch_types=[
            pltpu.VMEM_SHARED((NPAD, HALF), _F32),
            pltpu.VMEM((2 * G, CHUNK), jnp.int32),
            pltpu.VMEM((2 * G, CHUNK), jnp.int32),
            pltpu.VMEM((2 * G, CHUNK), _F32),
            pltpu.VMEM((2 * G, CHUNK, HALF), _F32),
            pltpu.VMEM((ZROWS, HALF), _F32),
            pltpu.SemaphoreType.DMA((2,)),
            pltpu.SemaphoreType.DMA((2,)),
            pltpu.SemaphoreType.DMA,
        ],
        name="sc_spmm3",
        compiler_params=_sc_compiler_params(),
    )
    return fn(xall, src_cat, dst_cat, val_cat)


_BLK = 2000
_GRID = N // _BLK


def _make_tc_layer(final):
    def body(*refs):
        if final:
            y3, x1, p00, p01, p02, wg, wgc, ce, s1c, s2c = refs[:10]
            outs = refs[10:]
            p0 = (p00, p01, p02)
        else:
            y3, wg, wgc, ce, s1c, s2c = refs[:6]
            outs = refs[6:]

        wgv = wg[...]
        wgcv = wgc[...]
        cev = ce[...]
        s1v = s1c[...]
        s2v = s2c[...]

        g = []
        for cc in range(C):
            e = jnp.concatenate([y3[cc], y3[C + cc]], axis=1)
            h = jnp.dot(e, wgv, preferred_element_type=_F32)
            h = h * cev[cc][None, :]
            h = _leaky(jnp.dot(h, wgcv, preferred_element_type=_F32))
            g.append(h)

        t = [jnp.tanh(jnp.dot(g[cc], s1v, preferred_element_type=_F32))
             for cc in range(C)]

        for i in range(C):
            s2row = s2v[i][None, :]
            l = [jnp.sum(t[cc][:, 32 * i:32 * i + 32] * s2row,
                         axis=1, keepdims=True) for cc in range(C)]
            m = jnp.maximum(jnp.maximum(l[0], l[1]), l[2])
            w = [jnp.exp(x - m) for x in l]
            tot = w[0] + w[1] + w[2]
            out = _leaky((w[0] * g[0] + w[1] * g[1] + w[2] * g[2]) / tot)
            if final:
                p1 = jnp.concatenate([x1[i], x1[C + i]], axis=1)
                outs[i][...] = (p0[i][...] + p1 + out) * (1.0 / 3.0)
            else:
                outs[0][i] = out[:, :HALF]
                outs[0][C + i] = out[:, HALF:]

    row_spec = lambda w: pl.BlockSpec((_BLK, w), lambda b: (b, 0))
    wt_spec = lambda a, b_: pl.BlockSpec((a, b_), lambda b: (0, 0))
    y3_spec = pl.BlockSpec((2 * C, _BLK, HALF), lambda b: (0, b, 0))

    wt_specs = [wt_spec(EMB, EMB), wt_spec(EMB, EMB),
                wt_spec(8, EMB), wt_spec(EMB, 96), wt_spec(8, 32)]
    if final:
        in_specs = [y3_spec, y3_spec] + [row_spec(EMB)] * 3 + wt_specs
        out_specs = [row_spec(EMB)] * 3
        out_shape = [jax.ShapeDtypeStruct((N, EMB), _F32)] * 3
    else:
        in_specs = [y3_spec] + wt_specs
        out_specs = [pl.BlockSpec((2 * C, _BLK, HALF), lambda b: (0, b, 0))]
        out_shape = [jax.ShapeDtypeStruct((2 * C, N, HALF), _F32)]

    return pl.pallas_call(
        body,
        grid=(_GRID,),
        in_specs=in_specs,
        out_specs=out_specs,
        out_shape=out_shape,
    )


def _cri_chain(ce0pad, wr0, wr1):
    def body(ce_ref, w0_ref, w1_ref, c1_ref, c2_ref, cm_ref):
        c0 = ce_ref[...]
        c1 = _leaky(jnp.dot(c0, w0_ref[...], preferred_element_type=_F32))
        c2 = _leaky(jnp.dot(c1, w1_ref[...], preferred_element_type=_F32))
        c1_ref[...] = c1
        c2_ref[...] = c2
        cm_ref[...] = (c0 + c1 + c2) * (1.0 / 3.0)

    return pl.pallas_call(
        body,
        out_shape=[jax.ShapeDtypeStruct((8, EMB), _F32)] * 3,
    )(ce0pad, wr0, wr1)


def kernel(adj_idx_0, adj_val_0, adj_idx_1, adj_val_1, adj_idx_2, adj_val_2,
           user_embedding, item_embedding, criterion_embedding, w_gcn,
           W_gc_0, W_gc_1, W_rel_0, W_rel_1, trans_s1, trans_s2):
    pad = EPAD - E
    adj = ((adj_idx_0, adj_val_0), (adj_idx_1, adj_val_1),
           (adj_idx_2, adj_val_2))
    src_cat = jnp.concatenate(
        [(jnp.pad(ai[1], (0, pad)) + i * N).reshape(EROWS, CHUNK)
         for i, (ai, _) in enumerate(adj)], axis=0)
    dst_cat = jnp.concatenate(
        [jnp.pad(ai[0], (0, pad)).reshape(EROWS, CHUNK)
         for ai, _ in adj], axis=0)
    val_cat = jnp.concatenate(
        [jnp.pad(av, (0, pad)).reshape(EROWS, CHUNK)
         for _, av in adj], axis=0)

    pre = jnp.concatenate([user_embedding, item_embedding], axis=0)
    p0 = [pre[:, i, :] for i in range(C)]
    xall0 = jnp.concatenate(
        [p0[i][:, :HALF] for i in range(C)]
        + [p0[i][:, HALF:] for i in range(C)], axis=0)

    ce0 = jnp.pad(criterion_embedding, ((0, 8 - C), (0, 0)))
    c1p, _c2p, cmp_ = _cri_chain(ce0, W_rel_0, W_rel_1)
    s1c = jnp.concatenate([trans_s1[i] for i in range(C)], axis=1)
    s2c = jnp.pad(jnp.squeeze(trans_s2, -1), ((0, 8 - C), (0, 0)))

    layer1 = _make_tc_layer(final=False)
    layer2 = _make_tc_layer(final=True)

    y1 = _sc_spmm3(xall0, src_cat, dst_cat, val_cat)
    y1 = y1.reshape(2 * C, NPAD, HALF)
    (x1,) = layer1(y1, w_gcn, W_gc_0, ce0, s1c, s2c)

    y2 = _sc_spmm3(x1.reshape(2 * C * N, HALF), src_cat, dst_cat, val_cat)
    y2 = y2.reshape(2 * C, NPAD, HALF)
    accs = layer2(y2, x1, p0[0], p0[1], p0[2],
                  w_gcn, W_gc_1, c1p, s1c, s2c)

    acc = jnp.stack(accs, axis=1)
    users = acc[:N_USERS]
    items = jnp.concatenate(
        [acc[N_USERS:], jnp.zeros((1, C, EMB), _F32)], axis=0)
    cris = tuple(cmp_[i:i + 1] for i in range(C))
    return (users, items) + cris
